# baseline (speedup 1.0000x reference)
import functools

import jax
import jax.numpy as jnp
from jax.experimental import pallas as pl
from jax.experimental.pallas import tpu as pltpu
from jax.experimental.pallas import tpu_sc as plsc

_N_CELLS = 16
_MPC = 8
_N_OFF = 9
_MAX_K = 16
_CHUNK = 256
_SUB = 256
_NCC = _N_CELLS * _N_CELLS
_NSLOT = _NCC * _MPC
_NCAND = _N_OFF * _MPC
_OFFSETS = ((-1, -1), (-1, 0), (-1, 1), (0, -1), (0, 0), (0, 1),
            (1, -1), (1, 0), (1, 1))

_HI = jax.lax.Precision.HIGHEST
_f32 = jnp.float32


def _gather_rows(table, idx):
    n = idx.shape[0]
    w = table.shape[1]
    f = w // 128
    sub = table.reshape(table.shape[0] * f, 128)
    idx7 = (idx[:, None] * f
            + jnp.arange(f, dtype=jnp.int32)[None, :]).reshape(1, n * f)
    window = 128
    mesh = plsc.VectorSubcoreMesh(core_axis_name="core",
                                  subcore_axis_name="subcore")

    @functools.partial(
        pl.kernel,
        out_type=jax.ShapeDtypeStruct((n * f, 128), table.dtype),
        mesh=mesh,
    )
    def kern(x_hbm, i_hbm, o_hbm):
        def body(i_vmem, o_vmem):
            pltpu.sync_copy(x_hbm.at[i_vmem.at[0]], o_vmem)

        pltpu.emit_pipeline(
            body,
            grid=(n * f // window,),
            in_specs=[pl.BlockSpec((1, window), index_map=lambda i: (0, i))],
            out_specs=[pl.BlockSpec((window, 128), index_map=lambda i: (i, 0))],
            core_axis_name=("core", "subcore"),
            dimension_semantics=(pltpu.PARALLEL,),
        )(i_hbm, o_hbm)

    return kern(sub, idx7).reshape(n, w)


def _dotbf(a, b, dims):
    return jax.lax.dot_general(a.astype(jnp.bfloat16), b.astype(jnp.bfloat16),
                               dims, preferred_element_type=_f32)


def _exp_gate(sc_masked, tau):
    raw = sc_masked - tau
    gate = jnp.where(raw > 0, raw, 1e-8 * jnp.exp(raw))
    return jnp.exp(gate) - 1.0


def _gate(eg):
    t = eg.shape[0]
    cnt_ge = jnp.zeros((t, _NCAND), _f32)
    a = eg[:, None, :]
    for blk in range(_N_OFF):
        b = eg[:, blk * _MPC:(blk + 1) * _MPC, None]
        cnt_ge = cnt_ge + jnp.sum((b >= a).astype(_f32), axis=1)
    thr = jnp.max(jnp.where(cnt_ge >= float(_MAX_K), eg, -jnp.inf),
                  axis=1, keepdims=True)
    egt = jnp.where(eg >= thr, eg, 0.0)
    gsum = jnp.sum(egt, axis=1, keepdims=True) + 1e-8
    gstr = jnp.tanh(jnp.max(egt, axis=1, keepdims=True))
    return egt / gsum * gstr


def _dawn_body(x_ref, ws_ref, nperm_ref, vgq_ref, vgv_ref, npq_ref, npv_ref,
               q_ref, k_ref, v_ref, ploss_ref):
    t = _SUB
    xb = x_ref[...]
    ws = ws_ref[...]
    dn0 = (((1,), (0,)), ((), ()))
    ypos = _dotbf(xb, ws[:, 0:4], dn0)
    y = jax.lax.dot_general(xb, ws, dn0,
                            preferred_element_type=_f32, precision=_HI)
    pos = jax.nn.sigmoid(ypos)
    cells = jnp.clip((pos * float(_N_CELLS)).astype(jnp.int32),
                     0, _N_CELLS - 1)
    scores = jax.lax.dot_general(xb, nperm_ref[...], (((1,), (1,)), ((), ())),
                                 preferred_element_type=_f32, precision=_HI)
    iota = jax.lax.broadcasted_iota(jnp.int32, (t, _NCC), 1)

    def nbr_cell(cx, cy, dx, dy):
        return (jnp.clip(cx + dx, 0, _N_CELLS - 1) * _N_CELLS
                + jnp.clip(cy + dy, 0, _N_CELLS - 1))

    def pool_gather(cx, cy, base, vg, npg):
        sc_cols, va_cols, npx_cols, npy_cols = [], [], [], []
        for (dx, dy) in _OFFSETS:
            oh = (nbr_cell(cx, cy, dx, dy) == iota).astype(_f32)
            va_cols.append(jnp.dot(oh, vg, preferred_element_type=_f32,
                                   precision=_HI))
            npxy = jnp.dot(oh, npg, preferred_element_type=_f32,
                           precision=_HI)
            npx_cols.append(npxy[:, 0:_MPC])
            npy_cols.append(npxy[:, _MPC:2 * _MPC])
            oht = jnp.concatenate([oh] * _MPC, axis=1)
            prod = oht * scores[:, base:base + _NSLOT]
            sc_cols.append(jnp.sum(prod.reshape(t, _MPC, _NCC),
                                   axis=2))
        return (jnp.concatenate(sc_cols, axis=1),
                jnp.concatenate(va_cols, axis=1),
                jnp.concatenate(npx_cols, axis=1),
                jnp.concatenate(npy_cols, axis=1))

    cxq, cyq = cells[:, 0:1], cells[:, 1:2]
    cxv, cyv = cells[:, 2:3], cells[:, 3:4]
    sc_q, va_q, npx_q, npy_q = pool_gather(cxq, cyq, 0, vgq_ref[...],
                                           npq_ref[...])
    sc_v, va_v, npx_v, npy_v = pool_gather(cxv, cyv, _NSLOT, vgv_ref[...],
                                           npv_ref[...])

    neg = jnp.float32(-1e9)
    scm_q = jnp.where(va_q > 0, sc_q, neg)
    scm_v = jnp.where(va_v > 0, sc_v, neg)
    gQ = _gate(_exp_gate(scm_q, y[:, 4:5]))
    gK = _gate(_exp_gate(scm_q, y[:, 5:6]))
    gV = _gate(_exp_gate(scm_v, y[:, 6:7]))
    wq = sc_q * gQ * va_q
    wk = sc_q * gK * va_q
    wv = sc_v * gV * va_v

    pdq = (pos[:, 0:1] - npx_q) ** 2 + (pos[:, 1:2] - npy_q) ** 2
    pdv = (pos[:, 2:3] - npx_v) ** 2 + (pos[:, 3:4] - npy_v) ** 2
    plq = jnp.sum(gQ * pdq * va_q)
    plv = jnp.sum(gV * pdv * va_v)
    mq = jnp.sum(va_q)
    mv = jnp.sum(va_v)
    li = jax.lax.broadcasted_iota(jnp.int32, (1, 1, 128), 2)
    vec = (jnp.where(li == 0, plq, 0.0) + jnp.where(li == 1, mq, 0.0)
           + jnp.where(li == 2, plv, 0.0) + jnp.where(li == 3, mv, 0.0))

    @pl.when(pl.program_id(0) % (_CHUNK // _SUB) == 0)
    def _():
        ploss_ref[...] = jnp.zeros_like(ploss_ref)

    ploss_ref[...] += vec

    def pool_scatter(ws, cx, cy):
        segs = [[jnp.zeros((t, _NCC), _f32) for _ in range(_MPC)]
                for _ in ws]
        for o, (dx, dy) in enumerate(_OFFSETS):
            oh = (nbr_cell(cx, cy, dx, dy) == iota).astype(_f32)
            for wi, w in enumerate(ws):
                for s in range(_MPC):
                    col = w[:, o * _MPC + s:o * _MPC + s + 1]
                    segs[wi][s] = segs[wi][s] + oh * col
        return [jnp.concatenate(sg, axis=1) for sg in segs]

    wq2, wk2 = pool_scatter([wq, wk], cxq, cyq)
    (wv2,) = pool_scatter([wv], cxv, cyv)
    nq = nperm_ref[0:_NSLOT, :]
    nv = nperm_ref[_NSLOT:2 * _NSLOT, :]
    dn = (((1,), (0,)), ((), ()))
    q_ref[...] = _dotbf(wq2, nq, dn)
    k_ref[...] = _dotbf(wk2, nq, dn)
    v_ref[...] = _dotbf(wv2, nv, dn)


def _attn_body(q_ref, k_ref, v_ref, wo_ref, o_ref):
    d_head = 64

    @pl.when(pl.program_id(1) == 0)
    def _():
        o_ref[...] = jnp.zeros_like(o_ref)

    acc = jnp.zeros_like(o_ref)
    for h in range(2):
        sl = slice(h * d_head, (h + 1) * d_head)
        qh = q_ref[:, sl]
        kh = k_ref[:, sl]
        logits = _dotbf(qh, kh, (((1,), (1,)), ((), ()))) * (
            1.0 / jnp.sqrt(_f32(d_head)))
        m = jnp.max(logits, axis=1, keepdims=True)
        p = jnp.exp(logits - m)
        s = jnp.sum(p, axis=1, keepdims=True)
        dn = (((1,), (0,)), ((), ()))
        ctx = _dotbf(p, v_ref[:, sl], dn) / s
        acc = acc + _dotbf(ctx, wo_ref[sl, :], dn)
    o_ref[...] += acc


def kernel(x, qk_neurons, v_neurons, npos_qk, npos_v, W_qk_pos, W_v_pos,
           w_tau_Q, w_tau_K, w_tau_V, expand_O_kernel, cell_map_qk,
           cell_map_v):
    B, S, D = x.shape
    n_chunks = S // _CHUNK
    n_heads = 12
    d_head = D // n_heads
    x2 = x.reshape(S, D)

    padw = 896 - (D + 2)
    aug_q = jnp.concatenate(
        [qk_neurons, npos_qk, jnp.zeros((qk_neurons.shape[0], padw), _f32)], 1)
    aug_v = jnp.concatenate(
        [v_neurons, npos_v, jnp.zeros((v_neurons.shape[0], padw), _f32)], 1)
    table = jnp.concatenate([aug_q, aug_v], axis=0)
    idx_q = jnp.maximum(cell_map_qk.T.reshape(-1), 0)
    idx_v = jnp.maximum(cell_map_v.T.reshape(-1), 0) + qk_neurons.shape[0]
    idx = jnp.concatenate([idx_q, idx_v]).astype(jnp.int32)
    gathered = _gather_rows(table, idx)
    nperm = gathered[:, :D]
    nps = gathered[:, D:D + 2].reshape(2, _MPC, _NCC, 2)
    npq = jnp.concatenate([nps[0, :, :, 0].T, nps[0, :, :, 1].T], axis=1)
    npv = jnp.concatenate([nps[1, :, :, 0].T, nps[1, :, :, 1].T], axis=1)
    vgq = (cell_map_qk >= 0).astype(_f32)
    vgv = (cell_map_v >= 0).astype(_f32)
    ws = jnp.concatenate(
        [W_qk_pos, W_v_pos, w_tau_Q[:, None], w_tau_K[:, None],
         w_tau_V[:, None], jnp.zeros((D, 1), _f32)], axis=1)

    q, k, v, pls = pl.pallas_call(
        _dawn_body,
        grid=(S // _SUB,),
        in_specs=[
            pl.BlockSpec((_SUB, D), lambda i: (i, 0)),
            pl.BlockSpec((D, 8), lambda i: (0, 0)),
            pl.BlockSpec((2 * _NSLOT, D), lambda i: (0, 0)),
            pl.BlockSpec((_NCC, _MPC), lambda i: (0, 0)),
            pl.BlockSpec((_NCC, _MPC), lambda i: (0, 0)),
            pl.BlockSpec((_NCC, 2 * _MPC), lambda i: (0, 0)),
            pl.BlockSpec((_NCC, 2 * _MPC), lambda i: (0, 0)),
        ],
        out_specs=[
            pl.BlockSpec((_SUB, D), lambda i: (i, 0)),
            pl.BlockSpec((_SUB, D), lambda i: (i, 0)),
            pl.BlockSpec((_SUB, D), lambda i: (i, 0)),
            pl.BlockSpec((1, 1, 128), lambda i: (i // (_CHUNK // _SUB), 0, 0)),
        ],
        out_shape=[
            jax.ShapeDtypeStruct((S, D), _f32),
            jax.ShapeDtypeStruct((S, D), _f32),
            jax.ShapeDtypeStruct((S, D), _f32),
            jax.ShapeDtypeStruct((n_chunks, 1, 128), _f32),
        ],
    )(x2, ws, nperm, vgq, vgv, npq, npv)

    qblk = 256
    out = pl.pallas_call(
        _attn_body,
        grid=(S // qblk, n_heads // 2),
        in_specs=[
            pl.BlockSpec((qblk, 2 * d_head), lambda i, j: (i, j)),
            pl.BlockSpec((S, 2 * d_head), lambda i, j: (0, j)),
            pl.BlockSpec((S, 2 * d_head), lambda i, j: (0, j)),
            pl.BlockSpec((2 * d_head, D), lambda i, j: (j, 0)),
        ],
        out_specs=pl.BlockSpec((qblk, D), lambda i, j: (i, 0)),
        out_shape=jax.ShapeDtypeStruct((S, D), _f32),
    )(q, k, v, expand_O_kernel)

    pls = pls.reshape(n_chunks, 128)
    pos_loss = jnp.sum(pls[:, 0] / (pls[:, 1] + 1e-8)
                       + pls[:, 2] / (pls[:, 3] + 1e-8)) / n_chunks
    return out.reshape(B, S, D), pos_loss.astype(_f32)


# speedup vs baseline: 1.0622x; 1.0622x over previous
import functools

import jax
import jax.numpy as jnp
from jax.experimental import pallas as pl
from jax.experimental.pallas import tpu as pltpu
from jax.experimental.pallas import tpu_sc as plsc

_N_CELLS = 16
_MPC = 8
_N_OFF = 9
_MAX_K = 16
_CHUNK = 256
_SUB = 256
_NCC = _N_CELLS * _N_CELLS
_NSLOT = _NCC * _MPC
_NCAND = _N_OFF * _MPC
_OFFSETS = ((-1, -1), (-1, 0), (-1, 1), (0, -1), (0, 0), (0, 1),
            (1, -1), (1, 0), (1, 1))

_HI = jax.lax.Precision.HIGHEST
_f32 = jnp.float32


def _gather_rows(table, idx):
    n = idx.shape[0]
    w = table.shape[1]
    f = w // 128
    sub = table.reshape(table.shape[0] * f, 128)
    idx7 = (idx[:, None] * f
            + jnp.arange(f, dtype=jnp.int32)[None, :]).reshape(1, n * f)
    window = 128
    mesh = plsc.VectorSubcoreMesh(core_axis_name="core",
                                  subcore_axis_name="subcore")

    @functools.partial(
        pl.kernel,
        out_type=jax.ShapeDtypeStruct((n * f, 128), table.dtype),
        mesh=mesh,
    )
    def kern(x_hbm, i_hbm, o_hbm):
        def body(i_vmem, o_vmem):
            pltpu.sync_copy(x_hbm.at[i_vmem.at[0]], o_vmem)

        pltpu.emit_pipeline(
            body,
            grid=(n * f // window,),
            in_specs=[pl.BlockSpec((1, window), index_map=lambda i: (0, i))],
            out_specs=[pl.BlockSpec((window, 128), index_map=lambda i: (i, 0))],
            core_axis_name=("core", "subcore"),
            dimension_semantics=(pltpu.PARALLEL,),
        )(i_hbm, o_hbm)

    return kern(sub, idx7).reshape(n, w)


def _dotbf(a, b, dims):
    return jax.lax.dot_general(a.astype(jnp.bfloat16), b.astype(jnp.bfloat16),
                               dims, preferred_element_type=_f32)


def _exp_gate(sc_masked, tau):
    raw = sc_masked - tau
    gate = jnp.where(raw > 0, raw, 1e-8 * jnp.exp(raw))
    return jnp.exp(gate) - 1.0


def _gate(eg):
    t = eg.shape[0]
    cnt_ge = jnp.zeros((t, _NCAND), _f32)
    a = eg[:, None, :]
    for blk in range(_N_OFF):
        b = eg[:, blk * _MPC:(blk + 1) * _MPC, None]
        cnt_ge = cnt_ge + jnp.sum((b >= a).astype(_f32), axis=1)
    thr = jnp.max(jnp.where(cnt_ge >= float(_MAX_K), eg, -jnp.inf),
                  axis=1, keepdims=True)
    egt = jnp.where(eg >= thr, eg, 0.0)
    gsum = jnp.sum(egt, axis=1, keepdims=True) + 1e-8
    gstr = jnp.tanh(jnp.max(egt, axis=1, keepdims=True))
    return egt / gsum * gstr


def _dawn_body(x_ref, ws_ref, nperm_ref, vgq_ref, vgv_ref, npq_ref, npv_ref,
               q_ref, k_ref, v_ref, ploss_ref):
    t = _SUB
    xb = x_ref[...]
    ws = ws_ref[...]
    dn0 = (((1,), (0,)), ((), ()))
    ypos = _dotbf(xb, ws[:, 0:4], dn0)
    y = jax.lax.dot_general(xb, ws, dn0,
                            preferred_element_type=_f32, precision=_HI)
    pos = jax.nn.sigmoid(ypos)
    cells = jnp.clip((pos * float(_N_CELLS)).astype(jnp.int32),
                     0, _N_CELLS - 1)
    scores = jax.lax.dot_general(xb, nperm_ref[...], (((1,), (1,)), ((), ())),
                                 preferred_element_type=_f32, precision=_HI)
    iota = jax.lax.broadcasted_iota(jnp.int32, (t, _NCC), 1)

    def nbr_cell(cx, cy, dx, dy):
        return (jnp.clip(cx + dx, 0, _N_CELLS - 1) * _N_CELLS
                + jnp.clip(cy + dy, 0, _N_CELLS - 1))

    def pool_gather(cx, cy, base, vg, npg):
        sc_cols, va_cols, npx_cols, npy_cols = [], [], [], []
        for (dx, dy) in _OFFSETS:
            oh = (nbr_cell(cx, cy, dx, dy) == iota).astype(_f32)
            va_cols.append(jnp.dot(oh, vg, preferred_element_type=_f32,
                                   precision=_HI))
            npxy = jnp.dot(oh, npg, preferred_element_type=_f32,
                           precision=_HI)
            npx_cols.append(npxy[:, 0:_MPC])
            npy_cols.append(npxy[:, _MPC:2 * _MPC])
            scs = [jnp.sum(oh * scores[:, base + s * _NCC:
                                       base + (s + 1) * _NCC],
                           axis=1, keepdims=True) for s in range(_MPC)]
            sc_cols.append(jnp.concatenate(scs, axis=1))
        return (jnp.concatenate(sc_cols, axis=1),
                jnp.concatenate(va_cols, axis=1),
                jnp.concatenate(npx_cols, axis=1),
                jnp.concatenate(npy_cols, axis=1))

    cxq, cyq = cells[:, 0:1], cells[:, 1:2]
    cxv, cyv = cells[:, 2:3], cells[:, 3:4]
    sc_q, va_q, npx_q, npy_q = pool_gather(cxq, cyq, 0, vgq_ref[...],
                                           npq_ref[...])
    sc_v, va_v, npx_v, npy_v = pool_gather(cxv, cyv, _NSLOT, vgv_ref[...],
                                           npv_ref[...])

    neg = jnp.float32(-1e9)
    scm_q = jnp.where(va_q > 0, sc_q, neg)
    scm_v = jnp.where(va_v > 0, sc_v, neg)
    gQ = _gate(_exp_gate(scm_q, y[:, 4:5]))
    gK = _gate(_exp_gate(scm_q, y[:, 5:6]))
    gV = _gate(_exp_gate(scm_v, y[:, 6:7]))
    wq = sc_q * gQ * va_q
    wk = sc_q * gK * va_q
    wv = sc_v * gV * va_v

    pdq = (pos[:, 0:1] - npx_q) ** 2 + (pos[:, 1:2] - npy_q) ** 2
    pdv = (pos[:, 2:3] - npx_v) ** 2 + (pos[:, 3:4] - npy_v) ** 2
    plq = jnp.sum(gQ * pdq * va_q)
    plv = jnp.sum(gV * pdv * va_v)
    mq = jnp.sum(va_q)
    mv = jnp.sum(va_v)
    li = jax.lax.broadcasted_iota(jnp.int32, (1, 1, 128), 2)
    vec = (jnp.where(li == 0, plq, 0.0) + jnp.where(li == 1, mq, 0.0)
           + jnp.where(li == 2, plv, 0.0) + jnp.where(li == 3, mv, 0.0))

    @pl.when(pl.program_id(0) % (_CHUNK // _SUB) == 0)
    def _():
        ploss_ref[...] = jnp.zeros_like(ploss_ref)

    ploss_ref[...] += vec

    def pool_scatter(ws, cx, cy):
        segs = [[jnp.zeros((t, _NCC), _f32) for _ in range(_MPC)]
                for _ in ws]
        for o, (dx, dy) in enumerate(_OFFSETS):
            oh = (nbr_cell(cx, cy, dx, dy) == iota).astype(_f32)
            for wi, w in enumerate(ws):
                for s in range(_MPC):
                    col = w[:, o * _MPC + s:o * _MPC + s + 1]
                    segs[wi][s] = segs[wi][s] + oh * col
        return [jnp.concatenate(sg, axis=1) for sg in segs]

    wq2, wk2 = pool_scatter([wq, wk], cxq, cyq)
    (wv2,) = pool_scatter([wv], cxv, cyv)
    nq = nperm_ref[0:_NSLOT, :]
    nv = nperm_ref[_NSLOT:2 * _NSLOT, :]
    dn = (((1,), (0,)), ((), ()))
    q_ref[...] = _dotbf(wq2, nq, dn)
    k_ref[...] = _dotbf(wk2, nq, dn)
    v_ref[...] = _dotbf(wv2, nv, dn)


def _attn_body(q_ref, k_ref, v_ref, wo_ref, o_ref):
    d_head = 64

    @pl.when(pl.program_id(1) == 0)
    def _():
        o_ref[...] = jnp.zeros_like(o_ref)

    acc = jnp.zeros_like(o_ref)
    for h in range(2):
        sl = slice(h * d_head, (h + 1) * d_head)
        qh = q_ref[:, sl]
        kh = k_ref[:, sl]
        logits = _dotbf(qh, kh, (((1,), (1,)), ((), ()))) * (
            1.0 / jnp.sqrt(_f32(d_head)))
        m = jnp.max(logits, axis=1, keepdims=True)
        p = jnp.exp(logits - m)
        s = jnp.sum(p, axis=1, keepdims=True)
        dn = (((1,), (0,)), ((), ()))
        ctx = _dotbf(p, v_ref[:, sl], dn) / s
        acc = acc + _dotbf(ctx, wo_ref[sl, :], dn)
    o_ref[...] += acc


def kernel(x, qk_neurons, v_neurons, npos_qk, npos_v, W_qk_pos, W_v_pos,
           w_tau_Q, w_tau_K, w_tau_V, expand_O_kernel, cell_map_qk,
           cell_map_v):
    B, S, D = x.shape
    n_chunks = S // _CHUNK
    n_heads = 12
    d_head = D // n_heads
    x2 = x.reshape(S, D)

    padw = 896 - (D + 2)
    aug_q = jnp.concatenate(
        [qk_neurons, npos_qk, jnp.zeros((qk_neurons.shape[0], padw), _f32)], 1)
    aug_v = jnp.concatenate(
        [v_neurons, npos_v, jnp.zeros((v_neurons.shape[0], padw), _f32)], 1)
    table = jnp.concatenate([aug_q, aug_v], axis=0)
    idx_q = jnp.maximum(cell_map_qk.T.reshape(-1), 0)
    idx_v = jnp.maximum(cell_map_v.T.reshape(-1), 0) + qk_neurons.shape[0]
    idx = jnp.concatenate([idx_q, idx_v]).astype(jnp.int32)
    gathered = _gather_rows(table, idx)
    nperm = gathered[:, :D]
    nps = gathered[:, D:D + 2].reshape(2, _MPC, _NCC, 2)
    npq = jnp.concatenate([nps[0, :, :, 0].T, nps[0, :, :, 1].T], axis=1)
    npv = jnp.concatenate([nps[1, :, :, 0].T, nps[1, :, :, 1].T], axis=1)
    vgq = (cell_map_qk >= 0).astype(_f32)
    vgv = (cell_map_v >= 0).astype(_f32)
    ws = jnp.concatenate(
        [W_qk_pos, W_v_pos, w_tau_Q[:, None], w_tau_K[:, None],
         w_tau_V[:, None], jnp.zeros((D, 1), _f32)], axis=1)

    q, k, v, pls = pl.pallas_call(
        _dawn_body,
        grid=(S // _SUB,),
        in_specs=[
            pl.BlockSpec((_SUB, D), lambda i: (i, 0)),
            pl.BlockSpec((D, 8), lambda i: (0, 0)),
            pl.BlockSpec((2 * _NSLOT, D), lambda i: (0, 0)),
            pl.BlockSpec((_NCC, _MPC), lambda i: (0, 0)),
            pl.BlockSpec((_NCC, _MPC), lambda i: (0, 0)),
            pl.BlockSpec((_NCC, 2 * _MPC), lambda i: (0, 0)),
            pl.BlockSpec((_NCC, 2 * _MPC), lambda i: (0, 0)),
        ],
        out_specs=[
            pl.BlockSpec((_SUB, D), lambda i: (i, 0)),
            pl.BlockSpec((_SUB, D), lambda i: (i, 0)),
            pl.BlockSpec((_SUB, D), lambda i: (i, 0)),
            pl.BlockSpec((1, 1, 128), lambda i: (i // (_CHUNK // _SUB), 0, 0)),
        ],
        out_shape=[
            jax.ShapeDtypeStruct((S, D), _f32),
            jax.ShapeDtypeStruct((S, D), _f32),
            jax.ShapeDtypeStruct((S, D), _f32),
            jax.ShapeDtypeStruct((n_chunks, 1, 128), _f32),
        ],
    )(x2, ws, nperm, vgq, vgv, npq, npv)

    qblk = 512
    out = pl.pallas_call(
        _attn_body,
        grid=(S // qblk, n_heads // 2),
        in_specs=[
            pl.BlockSpec((qblk, 2 * d_head), lambda i, j: (i, j)),
            pl.BlockSpec((S, 2 * d_head), lambda i, j: (0, j)),
            pl.BlockSpec((S, 2 * d_head), lambda i, j: (0, j)),
            pl.BlockSpec((2 * d_head, D), lambda i, j: (j, 0)),
        ],
        out_specs=pl.BlockSpec((qblk, D), lambda i, j: (i, 0)),
        out_shape=jax.ShapeDtypeStruct((S, D), _f32),
    )(q, k, v, expand_O_kernel)

    pls = pls.reshape(n_chunks, 128)
    pos_loss = jnp.sum(pls[:, 0] / (pls[:, 1] + 1e-8)
                       + pls[:, 2] / (pls[:, 3] + 1e-8)) / n_chunks
    return out.reshape(B, S, D), pos_loss.astype(_f32)
